# baseline (speedup 1.0000x reference)
import jax
import jax.numpy as jnp
from jax import lax
from jax.experimental import pallas as pl
from jax.experimental.pallas import tpu as pltpu
from jax.experimental.pallas import tpu_sc as plsc

_NW = 32
_CHUNK = 8
_NBUF = 4


def _sc_body(mid_hbm, tab_hbm, x_hbm, out_hbm,
             idx_v, row_v, buf_v, gsem,
             si0, si1, si2, si3, so0, so1, so2, so3):
    D = tab_hbm.shape[1]
    nvec = D // 16
    sin = [si0, si1, si2, si3]
    sout = [so0, so1, so2, so3]
    c = lax.axis_index("c")
    s = lax.axis_index("s")
    w = s * 2 + c
    rows_per_w = x_hbm.shape[0] // _NW
    n = rows_per_w // _CHUNK
    base_w = w * rows_per_w

    pltpu.sync_copy(mid_hbm, idx_v)
    pltpu.async_copy(tab_hbm.at[idx_v], row_v, gsem).wait()

    def copy_in(g, b):
        return pltpu.make_async_copy(
            x_hbm.at[pl.ds(base_w + g * _CHUNK, _CHUNK)], buf_v.at[b], sin[b])

    def copy_out(g, b):
        return pltpu.make_async_copy(
            buf_v.at[b], out_hbm.at[pl.ds(base_w + g * _CHUNK, _CHUNK)], sout[b])

    copy_in(0, 0).start()
    copy_in(1, 1).start()

    def outer(gg, _):
        for b in range(_NBUF):
            g = gg * _NBUF + b
            bp2 = (b + 2) % _NBUF

            @pl.when(g >= 2)
            def _():
                copy_out(g - 2, bp2).wait()

            @pl.when(g + 2 < n)
            def _():
                copy_in(g + 2, bp2).start()

            copy_in(g, b).wait()

            @plsc.parallel_loop(0, nvec, 1, unroll=4)
            def _(j):
                rvec = row_v[0, pl.ds(j * 16, 16)]
                for r in range(_CHUNK):
                    plsc.addupdate(buf_v.at[b, r, pl.ds(j * 16, 16)], rvec)

            copy_out(g, b).start()
        return 0

    lax.fori_loop(0, n // _NBUF, outer, 0)
    copy_out(n - 2, (n - 2) % _NBUF).wait()
    copy_out(n - 1, (n - 1) % _NBUF).wait()


def kernel(x, mod_emb_table, modality_id):
    B, S, D = x.shape
    R = B * S
    xf = x.reshape(R, D)
    mid = jnp.asarray(modality_id, jnp.int32).reshape(1)
    mesh = plsc.VectorSubcoreMesh(core_axis_name="c", subcore_axis_name="s")
    out = pl.kernel(
        _sc_body,
        mesh=mesh,
        out_type=jax.ShapeDtypeStruct((R, D), x.dtype),
        scratch_types=[
            pltpu.VMEM((1,), jnp.int32),
            pltpu.VMEM((1, D), x.dtype),
            pltpu.VMEM((_NBUF, _CHUNK, D), x.dtype),
            pltpu.SemaphoreType.DMA,
            pltpu.SemaphoreType.DMA,
            pltpu.SemaphoreType.DMA,
            pltpu.SemaphoreType.DMA,
            pltpu.SemaphoreType.DMA,
            pltpu.SemaphoreType.DMA,
            pltpu.SemaphoreType.DMA,
            pltpu.SemaphoreType.DMA,
            pltpu.SemaphoreType.DMA,
        ],
    )(mid, mod_emb_table, xf)
    return out.reshape(B, S, D)


# speedup vs baseline: 1.1552x; 1.1552x over previous
import jax
import jax.numpy as jnp
from jax import lax
from jax.experimental import pallas as pl
from jax.experimental.pallas import tpu as pltpu
from jax.experimental.pallas import tpu_sc as plsc

_BLOCK_R = 1024


def _scs_gather_body(mid_hbm, tab_hbm, row_hbm, mid_smem):
    pltpu.sync_copy(mid_hbm, mid_smem)
    m = mid_smem[0]
    pltpu.sync_copy(tab_hbm.at[pl.ds(m, 1)], row_hbm)


def _sc_gather(mid, mod_emb_table):
    D = mod_emb_table.shape[1]
    mesh = plsc.ScalarSubcoreMesh(axis_name="c", num_cores=1)
    return pl.kernel(
        _scs_gather_body,
        mesh=mesh,
        out_type=jax.ShapeDtypeStruct((1, D), mod_emb_table.dtype),
        scratch_types=[
            pltpu.SMEM((1,), jnp.int32),
        ],
    )(mid, mod_emb_table)


def _tc_add_body(x_ref, row_ref, o_ref):
    o_ref[...] = x_ref[...] + row_ref[...]


def kernel(x, mod_emb_table, modality_id):
    B, S, D = x.shape
    R = B * S
    xf = x.reshape(R, D)
    mid = jnp.asarray(modality_id, jnp.int32).reshape(1)
    row = _sc_gather(mid, mod_emb_table)
    out = pl.pallas_call(
        _tc_add_body,
        grid=(R // _BLOCK_R,),
        in_specs=[
            pl.BlockSpec((_BLOCK_R, D), lambda i: (i, 0)),
            pl.BlockSpec((1, D), lambda i: (0, 0)),
        ],
        out_specs=pl.BlockSpec((_BLOCK_R, D), lambda i: (i, 0)),
        out_shape=jax.ShapeDtypeStruct((R, D), x.dtype),
    )(xf, row)
    return out.reshape(B, S, D)
